# Initial kernel scaffold; baseline (speedup 1.0000x reference)
import jax
import jax.numpy as jnp
from jax.experimental import pallas as pl


def kernel(x, weights):
    raise NotImplementedError("write your pallas kernel here")


# speedup vs baseline: 19.7227x; 19.7227x over previous
import jax
import jax.numpy as jnp
from jax import lax
from jax.experimental import pallas as pl
from jax.experimental.pallas import tpu as pltpu
from jax.experimental.pallas import tpu_sc as plsc

_N = 128
_B = 4096
_NC, _NS, _L = 2, 16, 16
_NW = _NC * _NS
_BPW = _B // _NW
_CH = _BPW // _L
_INV_H = (_N - 1) / 2.0
_W0, _W1 = -7, 8
_BIG = 1e30
_OFFS = (-1, 0, 1, 2)


def _cubic(s):
    a = jnp.abs(s)
    a2 = a * a
    a3 = a2 * a
    r1 = 1.5 * a3 - 2.5 * a2 + 1.0
    r2 = -0.5 * a3 + 2.5 * a2 - 4.0 * a + 2.0
    return jnp.where(a < 1.0, r1, jnp.where(a < 2.0, r2, 0.0))


def _rne16(v):
    i = plsc.bitcast(v, jnp.int32)
    i = i + 0x7FFF + ((i >> 16) & 1)
    return plsc.bitcast(i & -65536, jnp.float32)


def _body(xs_hbm, ys_hbm, w_hbm, t_hbm, tsq_hbm, out_hbm,
          xb, yb, wt, ttv, tsq, tcb, ob, tsl):
    wid = lax.axis_index("s") * _NC + lax.axis_index("c")
    base = wid * _BPW
    pltpu.sync_copy(w_hbm, wt)
    pltpu.sync_copy(t_hbm, ttv)
    pltpu.sync_copy(tsq_hbm, tsq)
    pltpu.sync_copy(xs_hbm.at[pl.ds(base, _BPW)], xb)
    pltpu.sync_copy(ys_hbm.at[pl.ds(base, _BPW)], yb)
    for k in range(_N // _L):
        slk = pl.ds(k * _L, _L)
        tcb[slk] = _rne16(ttv[slk])

    def qv_body(j, _carry):
        sl = pl.ds(j * _L, _L)
        xv = xb[sl]
        yv = yb[sl]
        x0b = _rne16(xv)
        x1b = _rne16(yv)
        u = (xv + 1.0) * _INV_H
        v = (yv + 1.0) * _INV_H
        cx = u.astype(jnp.int32)
        cy = v.astype(jnp.int32)
        fxv = u - cx.astype(jnp.float32)
        fyv = v - cy.astype(jnp.float32)
        q0 = xv * xv
        q1 = yv * yv
        sx = q0 + q1

        cols, rows, okc, okr = [], [], [], []
        for d in _OFFS:
            c = cx + d
            r = cy + d
            okc.append((c >= 0) & (c <= _N - 1))
            okr.append((r >= 0) & (r <= _N - 1))
            cols.append(jnp.minimum(jnp.maximum(c, 0), _N - 1))
            rows.append(jnp.minimum(jnp.maximum(r, 0), _N - 1))

        cbxs = [plsc.load_gather(tcb, [cols[i]]) for i in range(4)]
        sqxs = [plsc.load_gather(tsq, [cols[i]]) for i in range(4)]
        cbys = [plsc.load_gather(tcb, [rows[i]]) for i in range(4)]
        sqys = [plsc.load_gather(tsq, [rows[i]]) for i in range(4)]
        for iy in range(4):
            p1 = x1b * cbys[iy]
            for ix in range(4):
                p0 = x0b * cbxs[ix]
                m = p0 + p1
                ssum = sx + (sqxs[ix] + sqys[iy])
                dst = ssum - (m + m)
                si = iy * 4 + ix
                tsl[pl.ds(si * _L, _L)] = dst
                ii = plsc.bitcast(dst, jnp.int32)
                up = ii + jnp.where(ii >= 0, 1, -1)
                tsl[pl.ds((16 + si) * _L, _L)] = plsc.bitcast(up, jnp.float32)

        cnt0 = tuple(jnp.zeros((_L,), jnp.float32) for _ in range(16))

        def dr_body(it, cnt):
            drv = it + _W0
            rr = cy + drv
            rok = (rr >= 0) & (rr <= _N - 1)
            rc = jnp.minimum(jnp.maximum(rr, 0), _N - 1)
            cby = plsc.load_gather(tcb, [rc])
            sqy = plsc.load_gather(tsq, [rc])
            p1 = x1b * cby
            tas, tbs = [], []
            for iy2 in range(4):
                sr = _OFFS[iy2]
                a_s = drv <= sr
                b_s = drv < sr
                for ix2 in range(4):
                    si = iy2 * 4 + ix2
                    tlt = tsl[pl.ds(si * _L, _L)]
                    tle = tsl[pl.ds((16 + si) * _L, _L)]
                    tas.append(jnp.where(a_s, tle, tlt))
                    tbs.append(jnp.where(b_s, tle, tlt))
            new = list(cnt)
            for dc in range(_W0, _W1 + 1):
                cc_ = cx + dc
                ok = rok & (cc_ >= 0) & (cc_ <= _N - 1)
                ccc = jnp.minimum(jnp.maximum(cc_, 0), _N - 1)
                cbx = plsc.load_gather(tcb, [ccc])
                sqx = plsc.load_gather(tsq, [ccc])
                m = (x0b * cbx) + p1
                ssum = sx + (sqx + sqy)
                dj = ssum - (m + m)
                djm = jnp.where(ok, dj, _BIG)
                for si in range(16):
                    scol = _OFFS[si % 4]
                    t_s = tas[si] if dc < scol else tbs[si]
                    new[si] = new[si] + jnp.where(djm < t_s, 1.0, 0.0)
            return tuple(new)

        cnt = lax.fori_loop(0, _W1 - _W0 + 1, dr_body, cnt0)

        wxs = [jnp.where(okc[i], _cubic(fxv - _OFFS[i]), 0.0) for i in range(4)]
        wys = [jnp.where(okr[i], _cubic(fyv - _OFFS[i]), 0.0) for i in range(4)]
        acc = jnp.zeros((_L,), jnp.float32)
        for iy in range(4):
            rbase = rows[iy] << 7
            for ix in range(4):
                si = iy * 4 + ix
                g = plsc.load_gather(wt, [rbase + cols[ix]])
                wgt = jnp.where(cnt[si] < 15.5, wys[iy] * wxs[ix], 0.0)
                acc = acc + g * wgt
        ob[sl] = acc
        return 0

    lax.fori_loop(0, _CH, qv_body, 0)
    pltpu.sync_copy(ob, out_hbm.at[pl.ds(base, _BPW)])


_mesh = plsc.VectorSubcoreMesh(
    core_axis_name="c", subcore_axis_name="s",
    num_cores=_NC, num_subcores=_NS)

_sc_call = pl.kernel(
    _body,
    out_type=jax.ShapeDtypeStruct((_B,), jnp.float32),
    mesh=_mesh,
    compiler_params=pltpu.CompilerParams(needs_layout_passes=False),
    scratch_types=[
        pltpu.VMEM((_BPW,), jnp.float32),
        pltpu.VMEM((_BPW,), jnp.float32),
        pltpu.VMEM((_N * _N,), jnp.float32),
        pltpu.VMEM((_N,), jnp.float32),
        pltpu.VMEM((_N,), jnp.float32),
        pltpu.VMEM((_N,), jnp.float32),
        pltpu.VMEM((_BPW,), jnp.float32),
        pltpu.VMEM((32 * _L,), jnp.float32),
    ],
)


def kernel(x, weights):
    xt = x.T
    xs = xt[0]
    ys = xt[1]
    t = jnp.linspace(-1.0, 1.0, _N)
    tsq = t * t
    wtab = weights.reshape(_N * _N)
    out = _sc_call(xs, ys, wtab, t, tsq)
    return (out, x)


# speedup vs baseline: 23.0687x; 1.1696x over previous
import jax
import jax.numpy as jnp
from jax import lax
from jax.experimental import pallas as pl
from jax.experimental.pallas import tpu as pltpu
from jax.experimental.pallas import tpu_sc as plsc

_N = 128
_B = 4096
_NC, _NS, _L = 2, 16, 16
_NW = _NC * _NS
_BPW = _B // _NW
_CH = _BPW // _L
_INV_H = (_N - 1) / 2.0
_W0, _W1 = -7, 8
_NWIN = _W1 - _W0 + 1
_BIG = 1e30
_OFFS = (-1, 0, 1, 2)


def _cubic(s):
    a = jnp.abs(s)
    a2 = a * a
    a3 = a2 * a
    r1 = 1.5 * a3 - 2.5 * a2 + 1.0
    r2 = -0.5 * a3 + 2.5 * a2 - 4.0 * a + 2.0
    return jnp.where(a < 1.0, r1, jnp.where(a < 2.0, r2, 0.0))


def _rne16(v):
    i = plsc.bitcast(v, jnp.int32)
    i = i + 0x7FFF + ((i >> 16) & 1)
    return plsc.bitcast(i & -65536, jnp.float32)


def _nextafter_up(v):
    ii = plsc.bitcast(v, jnp.int32)
    up = ii + jnp.where(ii >= 0, 1, -1)
    return plsc.bitcast(up, jnp.float32)


def _body(xs_hbm, ys_hbm, w_hbm, t_hbm, tsq_hbm, out_hbm,
          xb, yb, wt, ttv, tsq, tcb, ob, cbuf, wbuf):
    wid = lax.axis_index("s") * _NC + lax.axis_index("c")
    base = wid * _BPW
    pltpu.sync_copy(w_hbm, wt)
    pltpu.sync_copy(t_hbm, ttv)
    pltpu.sync_copy(tsq_hbm, tsq)
    pltpu.sync_copy(xs_hbm.at[pl.ds(base, _BPW)], xb)
    pltpu.sync_copy(ys_hbm.at[pl.ds(base, _BPW)], yb)
    for k in range(_N // _L):
        slk = pl.ds(k * _L, _L)
        tcb[slk] = _rne16(ttv[slk])

    def qv_body(j, _carry):
        sl = pl.ds(j * _L, _L)
        xv = xb[sl]
        yv = yb[sl]
        x0b = _rne16(xv)
        x1b = _rne16(yv)
        u = (xv + 1.0) * _INV_H
        v = (yv + 1.0) * _INV_H
        cx = u.astype(jnp.int32)
        cy = v.astype(jnp.int32)
        fxv = u - cx.astype(jnp.float32)
        fyv = v - cy.astype(jnp.float32)
        q0 = xv * xv
        q1 = yv * yv
        sx = q0 + q1

        for wi in range(_NWIN):
            dv = wi + _W0
            c = cx + dv
            okc_ = (c >= 0) & (c <= _N - 1)
            ccc = jnp.minimum(jnp.maximum(c, 0), _N - 1)
            cbx = plsc.load_gather(tcb, [ccc])
            sqx = plsc.load_gather(tsq, [ccc])
            cbuf[pl.ds(wi * _L, _L)] = x0b * cbx
            cbuf[pl.ds((_NWIN + wi) * _L, _L)] = jnp.where(okc_, sqx, _BIG)
            r = cy + dv
            okr_ = (r >= 0) & (r <= _N - 1)
            rc = jnp.minimum(jnp.maximum(r, 0), _N - 1)
            cby = plsc.load_gather(tcb, [rc])
            sqy = plsc.load_gather(tsq, [rc])
            cbuf[pl.ds((2 * _NWIN + wi) * _L, _L)] = x1b * cby
            cbuf[pl.ds((3 * _NWIN + wi) * _L, _L)] = jnp.where(okr_, sqy, _BIG)

        def row_b(i, _):
            p1 = cbuf[pl.ds((2 * _NWIN) * _L + i * _L, _L)]
            sqy = cbuf[pl.ds((3 * _NWIN) * _L + i * _L, _L)]
            for ci in range(_NWIN):
                a = cbuf[pl.ds(ci * _L, _L)]
                sqx = cbuf[pl.ds((_NWIN + ci) * _L, _L)]
                ssum = sx + (sqx + sqy)
                m = a + p1
                dst = ssum - (m + m)
                wbuf[pl.ds(i * (_NWIN * _L) + ci * _L, _L)] = dst
            return 0

        lax.fori_loop(0, _NWIN, row_b, 0)

        cnts = [None] * 16
        for g in range(4):
            sis = [4 * g + kk for kk in range(4)]
            srs = [_OFFS[si // 4] for si in sis]
            scs = [_OFFS[si % 4] for si in sis]
            tlt = []
            tle = []
            for kk in range(4):
                off = ((srs[kk] - _W0) * _NWIN + (scs[kk] - _W0)) * _L
                t0 = wbuf[pl.ds(off, _L)]
                tlt.append(t0)
                tle.append(_nextafter_up(t0))

            def row_c(i, cc, tlt=tlt, tle=tle, srs=srs, scs=scs):
                drv = i + _W0
                ta = [jnp.where(drv <= srs[kk], tle[kk], tlt[kk])
                      for kk in range(4)]
                tb = [jnp.where(drv < srs[kk], tle[kk], tlt[kk])
                      for kk in range(4)]
                new = list(cc)
                for ci in range(_NWIN):
                    d = wbuf[pl.ds(i * (_NWIN * _L) + ci * _L, _L)]
                    dc = ci + _W0
                    for kk in range(4):
                        t_s = ta[kk] if dc < scs[kk] else tb[kk]
                        new[kk] = new[kk] + jnp.where(d < t_s, 1.0, 0.0)
                return tuple(new)

            z = jnp.zeros((_L,), jnp.float32)
            out4 = lax.fori_loop(0, _NWIN, row_c, (z, z, z, z))
            for kk in range(4):
                cnts[sis[kk]] = out4[kk]

        cols, rows, okc, okr = [], [], [], []
        for d in _OFFS:
            c = cx + d
            r = cy + d
            okc.append((c >= 0) & (c <= _N - 1))
            okr.append((r >= 0) & (r <= _N - 1))
            cols.append(jnp.minimum(jnp.maximum(c, 0), _N - 1))
            rows.append(jnp.minimum(jnp.maximum(r, 0), _N - 1))
        wxs = [jnp.where(okc[i], _cubic(fxv - _OFFS[i]), 0.0) for i in range(4)]
        wys = [jnp.where(okr[i], _cubic(fyv - _OFFS[i]), 0.0) for i in range(4)]
        acc = jnp.zeros((_L,), jnp.float32)
        for iy in range(4):
            rbase = rows[iy] << 7
            for ix in range(4):
                si = iy * 4 + ix
                g = plsc.load_gather(wt, [rbase + cols[ix]])
                wgt = jnp.where(cnts[si] < 15.5, wys[iy] * wxs[ix], 0.0)
                acc = acc + g * wgt
        ob[sl] = acc
        return 0

    lax.fori_loop(0, _CH, qv_body, 0)
    pltpu.sync_copy(ob, out_hbm.at[pl.ds(base, _BPW)])


_mesh = plsc.VectorSubcoreMesh(
    core_axis_name="c", subcore_axis_name="s",
    num_cores=_NC, num_subcores=_NS)

_sc_call = pl.kernel(
    _body,
    out_type=jax.ShapeDtypeStruct((_B,), jnp.float32),
    mesh=_mesh,
    compiler_params=pltpu.CompilerParams(needs_layout_passes=False),
    scratch_types=[
        pltpu.VMEM((_BPW,), jnp.float32),
        pltpu.VMEM((_BPW,), jnp.float32),
        pltpu.VMEM((_N * _N,), jnp.float32),
        pltpu.VMEM((_N,), jnp.float32),
        pltpu.VMEM((_N,), jnp.float32),
        pltpu.VMEM((_N,), jnp.float32),
        pltpu.VMEM((_BPW,), jnp.float32),
        pltpu.VMEM((4 * _NWIN * _L,), jnp.float32),
        pltpu.VMEM((_NWIN * _NWIN * _L,), jnp.float32),
    ],
)


def kernel(x, weights):
    xt = x.T
    xs = xt[0]
    ys = xt[1]
    t = jnp.linspace(-1.0, 1.0, _N)
    tsq = t * t
    wtab = weights.reshape(_N * _N)
    out = _sc_call(xs, ys, wtab, t, tsq)
    return (out, x)
